# Initial kernel scaffold; baseline (speedup 1.0000x reference)
import jax
import jax.numpy as jnp
from jax.experimental import pallas as pl


def kernel(x, gate_w, sw1, sw2, sw3, rw1, rw2, rw3, expert_bias):
    raise NotImplementedError("write your pallas kernel here")


# speedup vs baseline: 1.1430x; 1.1430x over previous
import functools

import jax
import jax.numpy as jnp
from jax import lax
from jax.experimental import pallas as pl
from jax.experimental.pallas import tpu as pltpu
from jax.experimental.pallas import tpu_sc as plsc

E = 8
K = 2
D = 1024
F = 2048
N = 2048
CAP = N
BM = 128
NB = CAP // BM
SBM = 256
NW = 32


def _router_body(x_ref, gw_ref, eb_ref, probs_ref, pos_ref, w_ref, cnt_ref):
    x = x_ref[...]
    logits = lax.dot_general(x, gw_ref[...], (((1,), (1,)), ((), ())),
                             preferred_element_type=jnp.float32)
    logits = logits + eb_ref[...]
    m = jnp.max(logits, axis=1, keepdims=True)
    ex = jnp.exp(logits - m)
    probs = ex / jnp.sum(ex, axis=1, keepdims=True)
    probs_ref[...] = probs

    col = lax.broadcasted_iota(jnp.int32, (N, E), 1)
    p0 = jnp.max(probs, axis=1, keepdims=True)
    i0 = jnp.min(jnp.where(probs == p0, col, E), axis=1, keepdims=True)
    pm = jnp.where(col == i0, -jnp.inf, probs)
    p1 = jnp.max(pm, axis=1, keepdims=True)
    i1 = jnp.min(jnp.where(pm == p1, col, E), axis=1, keepdims=True)

    denom = p0 + p1
    w_ref[...] = jnp.concatenate([p0 / denom, p1 / denom], axis=1)

    oh0 = (col == i0).astype(jnp.float32)
    oh1 = (col == i1).astype(jnp.float32)

    CB = 256
    r = lax.broadcasted_iota(jnp.int32, (CB, CB), 0)
    c = lax.broadcasted_iota(jnp.int32, (CB, CB), 1)
    tri = (c < r).astype(jnp.float32)

    def excl_cumsum(oh):
        carry = jnp.zeros((1, E), jnp.float32)
        parts = []
        for j in range(N // CB):
            blk = lax.slice(oh, (j * CB, 0), ((j + 1) * CB, E))
            parts.append(lax.dot_general(tri, blk, (((1,), (0,)), ((), ())),
                                         preferred_element_type=jnp.float32)
                         + carry)
            carry = carry + jnp.sum(blk, axis=0, keepdims=True)
        return jnp.concatenate(parts, axis=0), carry

    r0, c0 = excl_cumsum(oh0)
    r1, c1 = excl_cumsum(oh1)
    rank0 = jnp.sum(r0 * oh0, axis=1, keepdims=True)
    rank1 = jnp.sum(r1 * oh1, axis=1, keepdims=True)
    c0_at_i1 = jnp.sum(oh1 * c0, axis=1, keepdims=True)
    pos0 = i0 * CAP + rank0.astype(jnp.int32)
    pos1 = i1 * CAP + (c0_at_i1 + rank1).astype(jnp.int32)
    pos_ref[...] = jnp.concatenate([pos0, pos1], axis=1)
    cnt_ref[...] = (c0 + c1).astype(jnp.int32)


def _router(x, gate_w, expert_bias):
    return pl.pallas_call(
        _router_body,
        out_shape=(
            jax.ShapeDtypeStruct((N, E), jnp.float32),
            jax.ShapeDtypeStruct((N, K), jnp.int32),
            jax.ShapeDtypeStruct((N, K), jnp.float32),
            jax.ShapeDtypeStruct((1, E), jnp.int32),
        ),
    )(x, gate_w, expert_bias)


def _dispatch_body(x_hbm, pos_hbm, xs_hbm, idx_v, rows_v, sem):
    wid = lax.axis_index("s") * 2 + lax.axis_index("c")
    tok_base = (wid % 16) * 128
    pltpu.sync_copy(pos_hbm.at[wid], idx_v)
    for j in range(8):
        pltpu.sync_copy(x_hbm.at[pl.ds(tok_base + j * 16, 16)], rows_v)
        pltpu.async_copy(rows_v, xs_hbm.at[idx_v[j]], sem).wait()


def _dispatch(x, pos_disp):
    mesh = plsc.VectorSubcoreMesh(core_axis_name="c", subcore_axis_name="s")
    return pl.kernel(
        _dispatch_body,
        out_type=jax.ShapeDtypeStruct((E * CAP, D), jnp.float32),
        mesh=mesh,
        scratch_types=[
            pltpu.VMEM((8, 16), jnp.int32),
            pltpu.VMEM((16, D), jnp.float32),
            pltpu.SemaphoreType.DMA,
        ],
    )(x, pos_disp)


def _ffn_compute(x_f32, w1b, w3b, w2b):
    xb = x_f32.astype(jnp.bfloat16)
    h1 = lax.dot_general(xb, w1b, (((1,), (1,)), ((), ())),
                         preferred_element_type=jnp.float32)
    h3 = lax.dot_general(xb, w3b, (((1,), (1,)), ((), ())),
                         preferred_element_type=jnp.float32)
    h = (h1 * jax.nn.sigmoid(h1)) * h3
    return lax.dot_general(h.astype(jnp.bfloat16), w2b, (((1,), (1,)), ((), ())),
                           preferred_element_type=jnp.float32)


def _shared_body(x_ref, w1_ref, w3_ref, w2_ref, o_ref, w1b, w3b, w2b):
    @pl.when(pl.program_id(0) == 0)
    def _():
        w1b[...] = w1_ref[...].astype(jnp.bfloat16)
        w3b[...] = w3_ref[...].astype(jnp.bfloat16)
        w2b[...] = w2_ref[...].astype(jnp.bfloat16)
    o_ref[...] = _ffn_compute(x_ref[...], w1b[...], w3b[...], w2b[...])


def _shared(x, sw1, sw3, sw2):
    return pl.pallas_call(
        _shared_body,
        grid=(N // SBM,),
        in_specs=[
            pl.BlockSpec((SBM, D), lambda b: (b, 0)),
            pl.BlockSpec((F, D), lambda b: (0, 0)),
            pl.BlockSpec((F, D), lambda b: (0, 0)),
            pl.BlockSpec((D, F), lambda b: (0, 0)),
        ],
        out_specs=pl.BlockSpec((SBM, D), lambda b: (b, 0)),
        out_shape=jax.ShapeDtypeStruct((N, D), jnp.float32),
        scratch_shapes=[
            pltpu.VMEM((F, D), jnp.bfloat16),
            pltpu.VMEM((F, D), jnp.bfloat16),
            pltpu.VMEM((D, F), jnp.bfloat16),
        ],
        compiler_params=pltpu.CompilerParams(
            vmem_limit_bytes=100 * 1024 * 1024),
    )(x, sw1, sw3, sw2)


def _routed_body(cnt_ref, xs_ref, w1_ref, w3_ref, w2_ref, ys_ref,
                 w1b, w3b, w2b):
    e = pl.program_id(0)
    b = pl.program_id(1)

    @pl.when(b == 0)
    def _():
        w1b[...] = w1_ref[0].astype(jnp.bfloat16)
        w3b[...] = w3_ref[0].astype(jnp.bfloat16)
        w2b[...] = w2_ref[0].astype(jnp.bfloat16)

    @pl.when(b * BM < cnt_ref[e])
    def _():
        ys_ref[...] = _ffn_compute(xs_ref[...], w1b[...], w3b[...], w2b[...])


def _clamped_row(e, b, cnt):
    nb_active = lax.max(1, lax.div(cnt[e] + BM - 1, BM))
    return e * NB + lax.min(b, nb_active - 1)


def _routed(counts, xs, rw1, rw3, rw2):
    grid_spec = pltpu.PrefetchScalarGridSpec(
        num_scalar_prefetch=1,
        grid=(E, NB),
        in_specs=[
            pl.BlockSpec((BM, D), lambda e, b, cnt: (_clamped_row(e, b, cnt), 0)),
            pl.BlockSpec((1, F, D), lambda e, b, cnt: (e, 0, 0)),
            pl.BlockSpec((1, F, D), lambda e, b, cnt: (e, 0, 0)),
            pl.BlockSpec((1, D, F), lambda e, b, cnt: (e, 0, 0)),
        ],
        out_specs=pl.BlockSpec((BM, D), lambda e, b, cnt: (_clamped_row(e, b, cnt), 0)),
        scratch_shapes=[
            pltpu.VMEM((F, D), jnp.bfloat16),
            pltpu.VMEM((F, D), jnp.bfloat16),
            pltpu.VMEM((D, F), jnp.bfloat16),
        ],
    )
    return pl.pallas_call(
        _routed_body,
        grid_spec=grid_spec,
        out_shape=jax.ShapeDtypeStruct((E * CAP, D), jnp.float32),
        compiler_params=pltpu.CompilerParams(
            vmem_limit_bytes=100 * 1024 * 1024),
    )(counts, xs, rw1, rw3, rw2)


def _combine_body(ys_hbm, sh_hbm, pos_hbm, wsp_hbm, out_hbm,
                  i0_v, i1_v, w0_v, w1_v, r0_v, r1_v, acc_v, sem):
    wid = lax.axis_index("s") * 2 + lax.axis_index("c")
    pltpu.sync_copy(pos_hbm.at[0, wid], i0_v)
    pltpu.sync_copy(pos_hbm.at[1, wid], i1_v)
    for j in range(4):
        base = wid * 64 + j * 16
        pltpu.async_copy(ys_hbm.at[i0_v[j]], r0_v, sem).wait()
        pltpu.async_copy(ys_hbm.at[i1_v[j]], r1_v, sem).wait()
        pltpu.sync_copy(wsp_hbm.at[0, wid, j], w0_v)
        pltpu.sync_copy(wsp_hbm.at[1, wid, j], w1_v)
        pltpu.sync_copy(sh_hbm.at[pl.ds(base, 16)], acc_v)

        def body(cc, carry):
            sl = pl.ds(pl.multiple_of(cc * 16, 16), 16)
            for i in range(16):
                acc_v[i, sl] = (acc_v[i, sl]
                                + w0_v[i, :] * r0_v[i, sl]
                                + w1_v[i, :] * r1_v[i, sl])
            return carry

        lax.fori_loop(0, D // 16, body, 0)
        pltpu.sync_copy(acc_v, out_hbm.at[pl.ds(base, 16)])


def _combine(ys, shared, pos_c, w_c):
    mesh = plsc.VectorSubcoreMesh(core_axis_name="c", subcore_axis_name="s")
    return pl.kernel(
        _combine_body,
        out_type=jax.ShapeDtypeStruct((N, D), jnp.float32),
        mesh=mesh,
        scratch_types=[
            pltpu.VMEM((4, 16), jnp.int32),
            pltpu.VMEM((4, 16), jnp.int32),
            pltpu.VMEM((16, 16), jnp.float32),
            pltpu.VMEM((16, 16), jnp.float32),
            pltpu.VMEM((16, D), jnp.float32),
            pltpu.VMEM((16, D), jnp.float32),
            pltpu.VMEM((16, D), jnp.float32),
            pltpu.SemaphoreType.DMA,
        ],
    )(ys, shared, pos_c, w_c)


def kernel(x, gate_w, sw1, sw2, sw3, rw1, rw2, rw3, expert_bias):
    Bx, Lx, Dx = x.shape
    xf = x.reshape(N, D)
    probs, pos_tk, w_tk, cnt2 = _router(xf, gate_w, expert_bias.reshape(1, E))
    counts = cnt2.reshape(E)
    pos_km = pos_tk.T
    pos_disp = pos_km.reshape(NW, 8, 16)
    pos_c = pos_km.reshape(K, NW, 4, 16)
    w_c = jnp.broadcast_to(w_tk.T[:, :, None], (K, N, 16)).reshape(K, NW, 4, 16, 16)

    xs = _dispatch(xf, pos_disp)
    shared = _shared(xf, sw1, sw3, sw2)
    ys = _routed(counts, xs, rw1, rw3, rw2)
    out = _combine(ys, shared, pos_c, w_c)
    return out.reshape(Bx, Lx, Dx), probs


# speedup vs baseline: 1.6375x; 1.4326x over previous
import functools

import jax
import jax.numpy as jnp
from jax import lax
from jax.experimental import pallas as pl
from jax.experimental.pallas import tpu as pltpu
from jax.experimental.pallas import tpu_sc as plsc

E = 8
K = 2
D = 1024
F = 2048
N = 2048
BM = 256
MAXB = (K * N) // BM + E
XR = MAXB * BM
SBM = 256
NW = 32


def _router_body(x_ref, gw_ref, eb_ref, probs_ref, pos_ref, w_ref,
                 bexp_ref, nb_ref):
    x = x_ref[...]
    logits = lax.dot_general(x, gw_ref[...], (((1,), (1,)), ((), ())),
                             preferred_element_type=jnp.float32)
    logits = logits + eb_ref[...]
    m = jnp.max(logits, axis=1, keepdims=True)
    ex = jnp.exp(logits - m)
    probs = ex / jnp.sum(ex, axis=1, keepdims=True)
    probs_ref[...] = probs

    col = lax.broadcasted_iota(jnp.int32, (N, E), 1)
    p0 = jnp.max(probs, axis=1, keepdims=True)
    i0 = jnp.min(jnp.where(probs == p0, col, E), axis=1, keepdims=True)
    pm = jnp.where(col == i0, -jnp.inf, probs)
    p1 = jnp.max(pm, axis=1, keepdims=True)
    i1 = jnp.min(jnp.where(pm == p1, col, E), axis=1, keepdims=True)

    denom = p0 + p1
    w_ref[...] = jnp.concatenate([p0 / denom, p1 / denom], axis=1)

    oh0 = (col == i0).astype(jnp.float32)
    oh1 = (col == i1).astype(jnp.float32)

    CB = 256
    r = lax.broadcasted_iota(jnp.int32, (CB, CB), 0)
    c = lax.broadcasted_iota(jnp.int32, (CB, CB), 1)
    tri = (c < r).astype(jnp.float32)

    def excl_cumsum(oh):
        carry = jnp.zeros((1, E), jnp.float32)
        parts = []
        for j in range(N // CB):
            blk = lax.slice(oh, (j * CB, 0), ((j + 1) * CB, E))
            parts.append(lax.dot_general(tri, blk, (((1,), (0,)), ((), ())),
                                         preferred_element_type=jnp.float32)
                         + carry)
            carry = carry + jnp.sum(blk, axis=0, keepdims=True)
        return jnp.concatenate(parts, axis=0), carry

    r0, c0 = excl_cumsum(oh0)
    r1, c1 = excl_cumsum(oh1)
    rank0 = jnp.sum(r0 * oh0, axis=1, keepdims=True)
    rank1 = jnp.sum(r1 * oh1, axis=1, keepdims=True)
    c0_at_i1 = jnp.sum(oh1 * c0, axis=1, keepdims=True)

    counts = c0 + c1
    padded = jnp.ceil(counts / BM) * BM
    er = lax.broadcasted_iota(jnp.int32, (E, E), 0)
    ec = lax.broadcasted_iota(jnp.int32, (E, E), 1)
    upper = (er < ec).astype(jnp.float32)
    aligned_off = lax.dot_general(padded, upper, (((1,), (0,)), ((), ())),
                                  preferred_element_type=jnp.float32)
    off_at_i0 = jnp.sum(oh0 * aligned_off, axis=1, keepdims=True)
    off_at_i1 = jnp.sum(oh1 * aligned_off, axis=1, keepdims=True)
    pos0 = off_at_i0.astype(jnp.int32) + rank0.astype(jnp.int32)
    pos1 = (off_at_i1 + c0_at_i1).astype(jnp.int32) + rank1.astype(jnp.int32)
    pos_ref[...] = jnp.concatenate([pos0, pos1], axis=1)

    ends = aligned_off + padded
    brow = (lax.broadcasted_iota(jnp.int32, (MAXB, E), 0) * BM).astype(jnp.float32)
    bexp = jnp.sum((brow >= ends).astype(jnp.int32), axis=1, keepdims=True)
    bexp_ref[...] = jnp.minimum(bexp, E - 1)
    nb_ref[...] = (jnp.sum(padded, axis=1, keepdims=True) / BM).astype(jnp.int32)


def _router(x, gate_w, expert_bias):
    return pl.pallas_call(
        _router_body,
        out_shape=(
            jax.ShapeDtypeStruct((N, E), jnp.float32),
            jax.ShapeDtypeStruct((N, K), jnp.int32),
            jax.ShapeDtypeStruct((N, K), jnp.float32),
            jax.ShapeDtypeStruct((MAXB, 1), jnp.int32),
            jax.ShapeDtypeStruct((1, 1), jnp.int32),
        ),
    )(x, gate_w, expert_bias)


def _dispatch_body(x_hbm, pos_hbm, xs_hbm, idx_v, rows_v, sem):
    wid = lax.axis_index("s") * 2 + lax.axis_index("c")
    tok_base = (wid % 16) * 128
    pltpu.sync_copy(pos_hbm.at[wid], idx_v)
    for j in range(8):
        pltpu.sync_copy(x_hbm.at[pl.ds(tok_base + j * 16, 16)], rows_v)
        pltpu.async_copy(rows_v, xs_hbm.at[idx_v[j]], sem).wait()


def _dispatch(x, pos_disp):
    mesh = plsc.VectorSubcoreMesh(core_axis_name="c", subcore_axis_name="s")
    return pl.kernel(
        _dispatch_body,
        out_type=jax.ShapeDtypeStruct((XR, D), jnp.float32),
        mesh=mesh,
        scratch_types=[
            pltpu.VMEM((8, 16), jnp.int32),
            pltpu.VMEM((16, D), jnp.float32),
            pltpu.SemaphoreType.DMA,
        ],
    )(x, pos_disp)


def _ffn_compute(x_f32, w1b, w3b, w2b):
    xb = x_f32.astype(jnp.bfloat16)
    h1 = lax.dot_general(xb, w1b, (((1,), (1,)), ((), ())),
                         preferred_element_type=jnp.float32)
    h3 = lax.dot_general(xb, w3b, (((1,), (1,)), ((), ())),
                         preferred_element_type=jnp.float32)
    h = (h1 * jax.nn.sigmoid(h1)) * h3
    return lax.dot_general(h.astype(jnp.bfloat16), w2b, (((1,), (1,)), ((), ())),
                           preferred_element_type=jnp.float32)


def _shared_body(x_ref, w1_ref, w3_ref, w2_ref, o_ref, w1b, w3b, w2b):
    @pl.when(pl.program_id(0) == 0)
    def _():
        w1b[...] = w1_ref[...].astype(jnp.bfloat16)
        w3b[...] = w3_ref[...].astype(jnp.bfloat16)
        w2b[...] = w2_ref[...].astype(jnp.bfloat16)
    o_ref[...] = _ffn_compute(x_ref[...], w1b[...], w3b[...], w2b[...])


def _shared(x, sw1, sw3, sw2):
    return pl.pallas_call(
        _shared_body,
        grid=(N // SBM,),
        in_specs=[
            pl.BlockSpec((SBM, D), lambda b: (b, 0)),
            pl.BlockSpec((F, D), lambda b: (0, 0)),
            pl.BlockSpec((F, D), lambda b: (0, 0)),
            pl.BlockSpec((D, F), lambda b: (0, 0)),
        ],
        out_specs=pl.BlockSpec((SBM, D), lambda b: (b, 0)),
        out_shape=jax.ShapeDtypeStruct((N, D), jnp.float32),
        scratch_shapes=[
            pltpu.VMEM((F, D), jnp.bfloat16),
            pltpu.VMEM((F, D), jnp.bfloat16),
            pltpu.VMEM((D, F), jnp.bfloat16),
        ],
        compiler_params=pltpu.CompilerParams(
            vmem_limit_bytes=100 * 1024 * 1024),
    )(x, sw1, sw3, sw2)


def _routed_body(bexp_ref, nb_ref, xs_ref, w1_ref, w3_ref, w2_ref, ys_ref,
                 w1b, w3b, w2b, w2f, sem):
    b = pl.program_id(0)
    active = b < nb_ref[0]
    e = bexp_ref[b]
    e_prev = bexp_ref[lax.max(b - 1, 0)]

    @pl.when(active & ((b == 0) | (e != e_prev)))
    def _():
        cp = pltpu.make_async_copy(w2_ref.at[e], w2f, sem)
        cp.start()
        w1b[...] = w1_ref[0].astype(jnp.bfloat16)
        w3b[...] = w3_ref[0].astype(jnp.bfloat16)
        cp.wait()
        w2b[...] = w2f[...].astype(jnp.bfloat16)

    @pl.when(active)
    def _():
        ys_ref[...] = _ffn_compute(xs_ref[...], w1b[...], w3b[...], w2b[...])


def _routed(bexp, nb, xs, rw1, rw3, rw2):
    def row_map(b, bexp_ref, nb_ref):
        return (lax.min(b, nb_ref[0] - 1), 0)

    def w_map(b, bexp_ref, nb_ref):
        return (bexp_ref[b], 0, 0)

    grid_spec = pltpu.PrefetchScalarGridSpec(
        num_scalar_prefetch=2,
        grid=(MAXB,),
        in_specs=[
            pl.BlockSpec((BM, D), row_map),
            pl.BlockSpec((1, F, D), w_map),
            pl.BlockSpec((1, F, D), w_map),
            pl.BlockSpec(memory_space=pl.ANY),
        ],
        out_specs=pl.BlockSpec((BM, D), row_map),
        scratch_shapes=[
            pltpu.VMEM((F, D), jnp.bfloat16),
            pltpu.VMEM((F, D), jnp.bfloat16),
            pltpu.VMEM((D, F), jnp.bfloat16),
            pltpu.VMEM((D, F), jnp.float32),
            pltpu.SemaphoreType.DMA,
        ],
    )
    return pl.pallas_call(
        _routed_body,
        grid_spec=grid_spec,
        out_shape=jax.ShapeDtypeStruct((XR, D), jnp.float32),
        compiler_params=pltpu.CompilerParams(
            vmem_limit_bytes=100 * 1024 * 1024),
    )(bexp, nb, xs, rw1, rw3, rw2)


def _combine_body(ys_hbm, sh_hbm, pos_hbm, wsp_hbm, out_hbm,
                  i0_v, i1_v, w0_v, w1_v, r0_v, r1_v, acc_v, sem):
    wid = lax.axis_index("s") * 2 + lax.axis_index("c")
    pltpu.sync_copy(pos_hbm.at[0, wid], i0_v)
    pltpu.sync_copy(pos_hbm.at[1, wid], i1_v)
    for j in range(4):
        base = wid * 64 + j * 16
        pltpu.async_copy(ys_hbm.at[i0_v[j]], r0_v, sem).wait()
        pltpu.async_copy(ys_hbm.at[i1_v[j]], r1_v, sem).wait()
        pltpu.sync_copy(wsp_hbm.at[0, wid, j], w0_v)
        pltpu.sync_copy(wsp_hbm.at[1, wid, j], w1_v)
        pltpu.sync_copy(sh_hbm.at[pl.ds(base, 16)], acc_v)

        def body(cc, carry):
            sl = pl.ds(pl.multiple_of(cc * 16, 16), 16)
            for i in range(16):
                acc_v[i, sl] = (acc_v[i, sl]
                                + w0_v[i, :] * r0_v[i, sl]
                                + w1_v[i, :] * r1_v[i, sl])
            return carry

        lax.fori_loop(0, D // 16, body, 0)
        pltpu.sync_copy(acc_v, out_hbm.at[pl.ds(base, 16)])


def _combine(ys, shared, pos_c, w_c):
    mesh = plsc.VectorSubcoreMesh(core_axis_name="c", subcore_axis_name="s")
    return pl.kernel(
        _combine_body,
        out_type=jax.ShapeDtypeStruct((N, D), jnp.float32),
        mesh=mesh,
        scratch_types=[
            pltpu.VMEM((4, 16), jnp.int32),
            pltpu.VMEM((4, 16), jnp.int32),
            pltpu.VMEM((16, 16), jnp.float32),
            pltpu.VMEM((16, 16), jnp.float32),
            pltpu.VMEM((16, D), jnp.float32),
            pltpu.VMEM((16, D), jnp.float32),
            pltpu.VMEM((16, D), jnp.float32),
            pltpu.SemaphoreType.DMA,
        ],
    )(ys, shared, pos_c, w_c)


def kernel(x, gate_w, sw1, sw2, sw3, rw1, rw2, rw3, expert_bias):
    Bx, Lx, Dx = x.shape
    xf = x.reshape(N, D)
    probs, pos_tk, w_tk, bexp2, nb2 = _router(xf, gate_w,
                                              expert_bias.reshape(1, E))
    bexp = bexp2.reshape(MAXB)
    nb = nb2.reshape(1)
    pos_km = pos_tk.T
    pos_disp = pos_km.reshape(NW, 8, 16)
    pos_c = pos_km.reshape(K, NW, 4, 16)
    w_c = jnp.broadcast_to(w_tk.T[:, :, None], (K, N, 16)).reshape(K, NW, 4, 16, 16)

    xs = _dispatch(xf, pos_disp)
    shared = _shared(xf, sw1, sw3, sw2)
    ys = _routed(bexp, nb, xs, rw1, rw3, rw2)
    out = _combine(ys, shared, pos_c, w_c)
    return out.reshape(Bx, Lx, Dx), probs


# speedup vs baseline: 1.7089x; 1.0436x over previous
import functools

import jax
import jax.numpy as jnp
from jax import lax
from jax.experimental import pallas as pl
from jax.experimental.pallas import tpu as pltpu
from jax.experimental.pallas import tpu_sc as plsc

E = 8
K = 2
D = 1024
F = 2048
N = 2048
BM = 256
MAXB = (K * N) // BM + E
XR = MAXB * BM
SBM = 512
NW = 32


def _router_body(x_ref, gw_ref, eb_ref, probs_ref, pos_ref, w_ref,
                 bexp_ref, nb_ref):
    x = x_ref[...]
    logits = lax.dot_general(x, gw_ref[...], (((1,), (1,)), ((), ())),
                             preferred_element_type=jnp.float32)
    logits = logits + eb_ref[...]
    m = jnp.max(logits, axis=1, keepdims=True)
    ex = jnp.exp(logits - m)
    probs = ex / jnp.sum(ex, axis=1, keepdims=True)
    probs_ref[...] = probs

    col = lax.broadcasted_iota(jnp.int32, (N, E), 1)
    p0 = jnp.max(probs, axis=1, keepdims=True)
    i0 = jnp.min(jnp.where(probs == p0, col, E), axis=1, keepdims=True)
    pm = jnp.where(col == i0, -jnp.inf, probs)
    p1 = jnp.max(pm, axis=1, keepdims=True)
    i1 = jnp.min(jnp.where(pm == p1, col, E), axis=1, keepdims=True)

    denom = p0 + p1
    w_ref[...] = jnp.concatenate([p0 / denom, p1 / denom], axis=1)

    oh0 = (col == i0).astype(jnp.float32)
    oh1 = (col == i1).astype(jnp.float32)

    CB = 256
    r = lax.broadcasted_iota(jnp.int32, (CB, CB), 0)
    c = lax.broadcasted_iota(jnp.int32, (CB, CB), 1)
    tri = (c < r).astype(jnp.float32)

    def excl_cumsum(oh):
        carry = jnp.zeros((1, E), jnp.float32)
        parts = []
        for j in range(N // CB):
            blk = lax.slice(oh, (j * CB, 0), ((j + 1) * CB, E))
            parts.append(lax.dot_general(tri, blk, (((1,), (0,)), ((), ())),
                                         preferred_element_type=jnp.float32)
                         + carry)
            carry = carry + jnp.sum(blk, axis=0, keepdims=True)
        return jnp.concatenate(parts, axis=0), carry

    r0, c0 = excl_cumsum(oh0)
    r1, c1 = excl_cumsum(oh1)
    rank0 = jnp.sum(r0 * oh0, axis=1, keepdims=True)
    rank1 = jnp.sum(r1 * oh1, axis=1, keepdims=True)
    c0_at_i1 = jnp.sum(oh1 * c0, axis=1, keepdims=True)

    counts = c0 + c1
    padded = jnp.ceil(counts / BM) * BM
    er = lax.broadcasted_iota(jnp.int32, (E, E), 0)
    ec = lax.broadcasted_iota(jnp.int32, (E, E), 1)
    upper = (er < ec).astype(jnp.float32)
    aligned_off = lax.dot_general(padded, upper, (((1,), (0,)), ((), ())),
                                  preferred_element_type=jnp.float32)
    off_at_i0 = jnp.sum(oh0 * aligned_off, axis=1, keepdims=True)
    off_at_i1 = jnp.sum(oh1 * aligned_off, axis=1, keepdims=True)
    pos0 = off_at_i0.astype(jnp.int32) + rank0.astype(jnp.int32)
    pos1 = (off_at_i1 + c0_at_i1).astype(jnp.int32) + rank1.astype(jnp.int32)
    pos_ref[...] = jnp.concatenate([pos0, pos1], axis=1)

    ends = aligned_off + padded
    brow = (lax.broadcasted_iota(jnp.int32, (MAXB, E), 0) * BM).astype(jnp.float32)
    bexp = jnp.sum((brow >= ends).astype(jnp.int32), axis=1, keepdims=True)
    bexp_ref[...] = jnp.minimum(bexp, E - 1)
    nb_ref[...] = (jnp.sum(padded, axis=1, keepdims=True) / BM).astype(jnp.int32)


def _router(x, gate_w, expert_bias):
    return pl.pallas_call(
        _router_body,
        out_shape=(
            jax.ShapeDtypeStruct((N, E), jnp.float32),
            jax.ShapeDtypeStruct((N, K), jnp.int32),
            jax.ShapeDtypeStruct((N, K), jnp.float32),
            jax.ShapeDtypeStruct((MAXB, 1), jnp.int32),
            jax.ShapeDtypeStruct((1, 1), jnp.int32),
        ),
    )(x, gate_w, expert_bias)


def _dispatch_body(x_hbm, pos_hbm, xs_hbm, idx_v, rows_v, sem):
    wid = lax.axis_index("s") * 2 + lax.axis_index("c")
    tok_base = (wid % 16) * 128
    pltpu.sync_copy(pos_hbm.at[wid], idx_v)
    for j in range(8):
        pltpu.sync_copy(x_hbm.at[pl.ds(tok_base + j * 16, 16)], rows_v)
        pltpu.async_copy(rows_v, xs_hbm.at[idx_v[j]], sem).wait()


def _dispatch(x, pos_disp):
    mesh = plsc.VectorSubcoreMesh(core_axis_name="c", subcore_axis_name="s")
    return pl.kernel(
        _dispatch_body,
        out_type=jax.ShapeDtypeStruct((XR, D), jnp.float32),
        mesh=mesh,
        scratch_types=[
            pltpu.VMEM((8, 16), jnp.int32),
            pltpu.VMEM((16, D), jnp.float32),
            pltpu.SemaphoreType.DMA,
        ],
    )(x, pos_disp)


def _ffn_compute(x_f32, w1b, w3b, w2b):
    xb = x_f32.astype(jnp.bfloat16)
    h1 = lax.dot_general(xb, w1b, (((1,), (1,)), ((), ())),
                         preferred_element_type=jnp.float32)
    h3 = lax.dot_general(xb, w3b, (((1,), (1,)), ((), ())),
                         preferred_element_type=jnp.float32)
    h = (h1 * jax.nn.sigmoid(h1)) * h3
    return lax.dot_general(h.astype(jnp.bfloat16), w2b, (((1,), (1,)), ((), ())),
                           preferred_element_type=jnp.float32)


def _shared_body(x_ref, w1_ref, w3_ref, w2_ref, o_ref, w1b, w3b, w2b):
    @pl.when(pl.program_id(0) == 0)
    def _():
        w1b[...] = w1_ref[...].astype(jnp.bfloat16)
        w3b[...] = w3_ref[...].astype(jnp.bfloat16)
        w2b[...] = w2_ref[...].astype(jnp.bfloat16)
    o_ref[...] = _ffn_compute(x_ref[...], w1b[...], w3b[...], w2b[...])


def _shared(x, sw1, sw3, sw2):
    return pl.pallas_call(
        _shared_body,
        grid=(N // SBM,),
        in_specs=[
            pl.BlockSpec((SBM, D), lambda b: (b, 0)),
            pl.BlockSpec((F, D), lambda b: (0, 0)),
            pl.BlockSpec((F, D), lambda b: (0, 0)),
            pl.BlockSpec((D, F), lambda b: (0, 0)),
        ],
        out_specs=pl.BlockSpec((SBM, D), lambda b: (b, 0)),
        out_shape=jax.ShapeDtypeStruct((N, D), jnp.float32),
        scratch_shapes=[
            pltpu.VMEM((F, D), jnp.bfloat16),
            pltpu.VMEM((F, D), jnp.bfloat16),
            pltpu.VMEM((D, F), jnp.bfloat16),
        ],
        compiler_params=pltpu.CompilerParams(
            vmem_limit_bytes=100 * 1024 * 1024),
    )(x, sw1, sw3, sw2)


def _routed_body(bexp_ref, nb_ref, xs_ref, w1_ref, w3_ref, w2_ref, ys_ref,
                 w1b, w3b, w2b, w2f, sem):
    b = pl.program_id(0)
    nbv = nb_ref[0]
    active = b < nbv
    e = bexp_ref[b]
    e_prev = bexp_ref[lax.max(b - 1, 0)]
    e_next = bexp_ref[lax.min(b + 1, MAXB - 1)]

    @pl.when(active & (b == 0))
    def _():
        pltpu.make_async_copy(w2_ref.at[e], w2f, sem).start()

    @pl.when(active & ((b == 0) | (e != e_prev)))
    def _():
        w1b[...] = w1_ref[0].astype(jnp.bfloat16)
        w3b[...] = w3_ref[0].astype(jnp.bfloat16)
        pltpu.make_async_copy(w2_ref.at[e], w2f, sem).wait()
        w2b[...] = w2f[...].astype(jnp.bfloat16)

    @pl.when(active & (e_next != e) & (b + 1 < nbv))
    def _():
        pltpu.make_async_copy(w2_ref.at[e_next], w2f, sem).start()

    @pl.when(active)
    def _():
        ys_ref[...] = _ffn_compute(xs_ref[...], w1b[...], w3b[...], w2b[...])


def _routed(bexp, nb, xs, rw1, rw3, rw2):
    def row_map(b, bexp_ref, nb_ref):
        return (lax.min(b, nb_ref[0] - 1), 0)

    def w_map(b, bexp_ref, nb_ref):
        return (bexp_ref[b], 0, 0)

    grid_spec = pltpu.PrefetchScalarGridSpec(
        num_scalar_prefetch=2,
        grid=(MAXB,),
        in_specs=[
            pl.BlockSpec((BM, D), row_map),
            pl.BlockSpec((1, F, D), w_map),
            pl.BlockSpec((1, F, D), w_map),
            pl.BlockSpec(memory_space=pl.ANY),
        ],
        out_specs=pl.BlockSpec((BM, D), row_map),
        scratch_shapes=[
            pltpu.VMEM((F, D), jnp.bfloat16),
            pltpu.VMEM((F, D), jnp.bfloat16),
            pltpu.VMEM((D, F), jnp.bfloat16),
            pltpu.VMEM((D, F), jnp.float32),
            pltpu.SemaphoreType.DMA,
        ],
    )
    return pl.pallas_call(
        _routed_body,
        grid_spec=grid_spec,
        out_shape=jax.ShapeDtypeStruct((XR, D), jnp.float32),
        compiler_params=pltpu.CompilerParams(
            vmem_limit_bytes=100 * 1024 * 1024),
    )(bexp, nb, xs, rw1, rw3, rw2)


def _combine_body(ys_hbm, sh_hbm, pos_hbm, wsp_hbm, out_hbm,
                  idx_v, w0_v, w1_v, rr_v, acc_v, sem0, sem1):
    wid = lax.axis_index("s") * 2 + lax.axis_index("c")
    pltpu.sync_copy(pos_hbm.at[wid], idx_v)
    sems = (sem0, sem1)
    pltpu.make_async_copy(ys_hbm.at[idx_v.at[0]], rr_v.at[0], sems[0]).start()
    for j in range(4):
        jj = j % 2
        base = wid * 64 + j * 16
        if j + 1 < 4:
            pltpu.make_async_copy(ys_hbm.at[idx_v.at[j + 1]], rr_v.at[1 - jj],
                                  sems[1 - jj]).start()
        pltpu.sync_copy(sh_hbm.at[pl.ds(base, 16)], acc_v)
        pltpu.sync_copy(wsp_hbm.at[0, wid, j], w0_v)
        pltpu.sync_copy(wsp_hbm.at[1, wid, j], w1_v)
        pltpu.make_async_copy(ys_hbm.at[idx_v.at[j]], rr_v.at[jj],
                              sems[jj]).wait()

        def body(cc, carry):
            sl = pl.ds(pl.multiple_of(cc * 16, 16), 16)
            for i in range(16):
                acc_v[i, sl] = (acc_v[i, sl]
                                + w0_v[i, :] * rr_v[jj, i, sl]
                                + w1_v[i, :] * rr_v[jj, 16 + i, sl])
            return carry

        lax.fori_loop(0, D // 16, body, 0)
        pltpu.sync_copy(acc_v, out_hbm.at[pl.ds(base, 16)])


def _combine(ys, shared, pos_c, w_c):
    mesh = plsc.VectorSubcoreMesh(core_axis_name="c", subcore_axis_name="s")
    return pl.kernel(
        _combine_body,
        out_type=jax.ShapeDtypeStruct((N, D), jnp.float32),
        mesh=mesh,
        scratch_types=[
            pltpu.VMEM((4, 32), jnp.int32),
            pltpu.VMEM((16, 16), jnp.float32),
            pltpu.VMEM((16, 16), jnp.float32),
            pltpu.VMEM((2, 32, D), jnp.float32),
            pltpu.VMEM((16, D), jnp.float32),
            pltpu.SemaphoreType.DMA,
            pltpu.SemaphoreType.DMA,
        ],
    )(ys, shared, pos_c, w_c)


def kernel(x, gate_w, sw1, sw2, sw3, rw1, rw2, rw3, expert_bias):
    Bx, Lx, Dx = x.shape
    xf = x.reshape(N, D)
    probs, pos_tk, w_tk, bexp2, nb2 = _router(xf, gate_w,
                                              expert_bias.reshape(1, E))
    bexp = bexp2.reshape(MAXB)
    nb = nb2.reshape(1)
    pos_km = pos_tk.T
    pos_disp = pos_km.reshape(NW, 8, 16)
    pos_c = jnp.concatenate([pos_km[0].reshape(NW, 4, 16),
                             pos_km[1].reshape(NW, 4, 16)],
                            axis=2)
    w_c = jnp.broadcast_to(w_tk.T[:, :, None], (K, N, 16)).reshape(K, NW, 4, 16, 16)

    xs = _dispatch(xf, pos_disp)
    shared = _shared(xf, sw1, sw3, sw2)
    ys = _routed(bexp, nb, xs, rw1, rw3, rw2)
    out = _combine(ys, shared, pos_c, w_c)
    return out.reshape(Bx, Lx, Dx), probs
